# Initial kernel scaffold; baseline (speedup 1.0000x reference)
import jax
import jax.numpy as jnp
from jax.experimental import pallas as pl


def kernel(x, edge_index0, edge_index1, pos_edges, neg_edges, emb, W_self0, W_neigh0, b0, W_self1, W_neigh1, b1, dec_W1, dec_b1, dec_W2, dec_b2, dec_W3, dec_b3):
    raise NotImplementedError("write your pallas kernel here")


# speedup vs baseline: 3.4469x; 3.4469x over previous
import functools

import jax
import jax.numpy as jnp
from jax import lax
from jax.experimental import pallas as pl
from jax.experimental.pallas import tpu as pltpu
from jax.experimental.pallas import tpu_sc as plsc

NC = 2
NS = 16
NW = NC * NS

H = 128
CH = 80


def _mesh():
    return plsc.VectorSubcoreMesh(core_axis_name="c", subcore_axis_name="s")


def _make_row_gather(n_idx, chunk=128):
    per_w = n_idx // NW
    n_chunks = per_w // chunk
    assert per_w * NW == n_idx and n_chunks * chunk == per_w

    @functools.partial(
        pl.kernel,
        out_type=jax.ShapeDtypeStruct((n_idx, H), jnp.float32),
        mesh=_mesh(),
        scratch_types=[
            pltpu.VMEM((chunk,), jnp.int32),
            pltpu.VMEM((chunk, H), jnp.float32),
            pltpu.SemaphoreType.DMA,
        ],
    )
    def k(table_hbm, idx_hbm, out_hbm, idx_v, rows_v, sem):
        wid = lax.axis_index("s") * NC + lax.axis_index("c")
        base = wid * per_w

        def body(c, carry):
            off = base + c * chunk
            pltpu.sync_copy(idx_hbm.at[pl.ds(off, chunk)], idx_v)
            pltpu.async_copy(table_hbm.at[idx_v], rows_v, sem).wait()
            pltpu.sync_copy(rows_v, out_hbm.at[pl.ds(off, chunk)])
            return carry

        lax.fori_loop(0, n_chunks, body, 0)

    return k


def _make_seg_sum(n_acc, n_edges, chunk=CH):
    per_w = n_edges // NW
    n_chunks = per_w // chunk
    rpt = n_acc // NS
    slabs = rpt // chunk
    assert per_w * NW == n_edges and n_chunks * chunk == per_w
    assert rpt * NS == n_acc and slabs * chunk == rpt and rpt % 8 == 0

    @functools.partial(
        pl.kernel,
        out_type=jax.ShapeDtypeStruct((NC, n_acc, H), jnp.float32),
        mesh=_mesh(),
        scratch_types=[
            pltpu.VMEM_SHARED((n_acc, H), jnp.float32),
            pltpu.VMEM((chunk,), jnp.int32),
            pltpu.VMEM((chunk,), jnp.int32),
            pltpu.VMEM((chunk, H), jnp.float32),
            pltpu.SemaphoreType.DMA,
        ],
    )
    def k(h_hbm, src_hbm, dst_hbm, zrow_hbm, acc_hbm,
          acc_s, sidx, didx, rows_v, sem):
        cc = lax.axis_index("c")
        sc = lax.axis_index("s")
        wid = sc * NC + cc
        r0 = sc * rpt

        pltpu.sync_copy(zrow_hbm, rows_v)
        for j in range(slabs):
            pltpu.sync_copy(rows_v, acc_s.at[pl.ds(r0 + j * chunk, chunk)])
        plsc.subcore_barrier()

        base = wid * per_w

        def body(c, carry):
            off = base + c * chunk
            pltpu.sync_copy(src_hbm.at[pl.ds(off, chunk)], sidx)
            pltpu.sync_copy(dst_hbm.at[pl.ds(off, chunk)], didx)
            pltpu.async_copy(h_hbm.at[sidx], rows_v, sem).wait()
            pltpu.sync_copy(rows_v, acc_s.at[didx], add=True)
            return carry

        lax.fori_loop(0, n_chunks, body, 0)
        plsc.subcore_barrier()

        for j in range(slabs):
            pltpu.sync_copy(acc_s.at[pl.ds(r0 + j * chunk, chunk)], rows_v)
            pltpu.sync_copy(rows_v, acc_hbm.at[cc, pl.ds(r0 + j * chunk, chunk)])

    return k


def _make_counts(n_acc, n_edges, chunk=CH):
    per_w = n_edges // NW
    n_chunks = per_w // chunk
    rpt = n_acc // NS
    slabs = rpt // chunk
    assert per_w * NW == n_edges and n_chunks * chunk == per_w
    assert rpt * NS == n_acc and slabs * chunk == rpt and rpt % 8 == 0

    @functools.partial(
        pl.kernel,
        out_type=jax.ShapeDtypeStruct((2, NC, n_acc, H), jnp.float32),
        mesh=_mesh(),
        scratch_types=[
            pltpu.VMEM_SHARED((n_acc, H), jnp.float32),
            pltpu.VMEM((chunk,), jnp.int32),
            pltpu.VMEM((chunk, H), jnp.float32),
            pltpu.VMEM((chunk, H), jnp.float32),
        ],
    )
    def k(dst0_hbm, dst1_hbm, zrow_hbm, ones_hbm, cnt_hbm,
          cnt_s, didx, ones_v, stage_v):
        cc = lax.axis_index("c")
        sc = lax.axis_index("s")
        wid = sc * NC + cc
        r0 = sc * rpt
        base = wid * per_w

        pltpu.sync_copy(zrow_hbm, stage_v)
        pltpu.sync_copy(ones_hbm, ones_v)

        for layer, dst_hbm in ((0, dst0_hbm), (1, dst1_hbm)):
            pltpu.sync_copy(zrow_hbm, stage_v)
            for j in range(slabs):
                pltpu.sync_copy(stage_v, cnt_s.at[pl.ds(r0 + j * chunk, chunk)])
            plsc.subcore_barrier()

            def body(c, carry):
                off = base + c * chunk
                pltpu.sync_copy(dst_hbm.at[pl.ds(off, chunk)], didx)
                pltpu.sync_copy(ones_v, cnt_s.at[didx], add=True)
                return carry

            lax.fori_loop(0, n_chunks, body, 0)
            plsc.subcore_barrier()

            for j in range(slabs):
                pltpu.sync_copy(cnt_s.at[pl.ds(r0 + j * chunk, chunk)], stage_v)
                pltpu.sync_copy(
                    stage_v,
                    cnt_hbm.at[layer, cc, pl.ds(r0 + j * chunk, chunk)])

    return k


def _sage_layer_tc(h, accA, accB, cntA, cntB, W_self, W_neigh, b, relu):
    n = h.shape[0]
    blk = 1000
    grid = n // blk

    def body(h_ref, aA_ref, aB_ref, cA_ref, cB_ref, ws_ref, wn_ref, b_ref,
             out_ref):
        cnt = cA_ref[:, :1] + cB_ref[:, :1]
        m = (aA_ref[...] + aB_ref[...]) / jnp.maximum(cnt, 1.0)
        y = (jnp.dot(h_ref[...], ws_ref[...],
                     preferred_element_type=jnp.float32)
             + jnp.dot(m, wn_ref[...], preferred_element_type=jnp.float32)
             + b_ref[...])
        if relu:
            y = jnp.maximum(y, 0.0)
        out_ref[...] = y

    row_spec = pl.BlockSpec((blk, H), lambda i: (i, 0))
    full = pl.BlockSpec((H, H), lambda i: (0, 0))
    bias = pl.BlockSpec((1, H), lambda i: (0, 0))
    return pl.pallas_call(
        body,
        grid=(grid,),
        in_specs=[row_spec, row_spec, row_spec, row_spec, row_spec,
                  full, full, bias],
        out_specs=row_spec,
        out_shape=jax.ShapeDtypeStruct((n, H), jnp.float32),
    )(h, accA, accB, cntA, cntB, W_self, W_neigh, b.reshape(1, H))


def _decoder_tc(rows, p, W1, b1, W2, b2, W3p, b3p):
    blk = 1024
    nblk = p // blk
    grid = 2 * nblk

    def body(a_ref, b_ref, w1_ref, b1_ref, w2_ref, b2_ref, w3_ref, b3_ref,
             out_ref):
        z = a_ref[...] * b_ref[...]
        z = jnp.maximum(
            jnp.dot(z, w1_ref[...], preferred_element_type=jnp.float32)
            + b1_ref[...], 0.0)
        z = jnp.maximum(
            jnp.dot(z, w2_ref[...], preferred_element_type=jnp.float32)
            + b2_ref[...], 0.0)
        out_ref[...] = (jnp.dot(z, w3_ref[...],
                                preferred_element_type=jnp.float32)
                        + b3_ref[...])

    def a_map(g):
        return (jnp.where(g < nblk, g, g + nblk), 0)

    def b_map(g):
        return (jnp.where(g < nblk, g + nblk, g + 2 * nblk), 0)

    full = pl.BlockSpec((H, H), lambda g: (0, 0))
    bias = pl.BlockSpec((1, H), lambda g: (0, 0))
    return pl.pallas_call(
        body,
        grid=(grid,),
        in_specs=[pl.BlockSpec((blk, H), a_map),
                  pl.BlockSpec((blk, H), b_map),
                  full, bias, full, bias, full, bias],
        out_specs=pl.BlockSpec((blk, H), lambda g: (g, 0)),
        out_shape=jax.ShapeDtypeStruct((2 * p, H), jnp.float32),
    )(rows, rows, W1, b1.reshape(1, H), W2, b2.reshape(1, H), W3p, b3p)


def kernel(x, edge_index0, edge_index1, pos_edges, neg_edges, emb,
           W_self0, W_neigh0, b0, W_self1, W_neigh1, b1,
           dec_W1, dec_b1, dec_W2, dec_b2, dec_W3, dec_b3):
    n = x.shape[0]
    e = edge_index0.shape[1]
    p = pos_edges.shape[1]

    x = x.astype(jnp.int32)
    ei0 = edge_index0.astype(jnp.int32)
    ei1 = edge_index1.astype(jnp.int32)

    n_pad = ((n + 128 * NW - 1) // (128 * NW)) * (128 * NW)
    x_pad = jnp.pad(x, (0, n_pad - n))
    h0 = _make_row_gather(n_pad)(emb, x_pad)[:n]

    qh = CH * NS
    n_acc = ((n + qh - 1) // qh) * qh
    zrow = jnp.zeros((CH, H), jnp.float32)
    ones = jnp.ones((CH, H), jnp.float32)

    cnt = _make_counts(n_acc, e)(ei0[1], ei1[1], zrow, ones)
    seg = _make_seg_sum(n_acc, e)

    acc0 = seg(h0, ei0[0], ei0[1], zrow)
    h1 = _sage_layer_tc(h0, acc0[0, :n], acc0[1, :n], cnt[0, 0, :n],
                        cnt[0, 1, :n], W_self0, W_neigh0, b0, relu=True)

    acc1 = seg(h1, ei1[0], ei1[1], zrow)
    h2 = _sage_layer_tc(h1, acc1[0, :n], acc1[1, :n], cnt[1, 0, :n],
                        cnt[1, 1, :n], W_self1, W_neigh1, b1, relu=False)

    idx_all = jnp.concatenate([pos_edges[0], pos_edges[1],
                               neg_edges[0], neg_edges[1]]).astype(jnp.int32)
    rows = _make_row_gather(4 * p)(h2, idx_all)

    W3p = jnp.pad(dec_W3, ((0, 0), (0, H - 1)))
    b3p = jnp.pad(dec_b3, (0, H - 1)).reshape(1, H)
    out = _decoder_tc(rows, p, dec_W1, dec_b1, dec_W2, dec_b2, W3p, b3p)

    h_pos = out[:p, :1]
    h_neg = out[p:2 * p, :1]
    return (h_pos, h_neg)
